# baseline (speedup 1.0000x reference)
import functools
import jax
import jax.numpy as jnp
from jax import lax
from jax.experimental import pallas as pl
from jax.experimental.pallas import tpu as pltpu
from jax.experimental.pallas import tpu_sc as plsc

N = 25000
E = 100000
H = 128
NTYPES = 8
NC = 2
NS = 16
CHUNK = 112
KCH = 56
EPAD = NS * KCH * CHUNK
R = 25024
ROWS_PER_TILE = R // NS
ZROWS = 34
DUMMY = N

@functools.cache
def _mesh():
    return plsc.VectorSubcoreMesh(core_axis_name="c", subcore_axis_name="s",
                                  num_cores=NC, num_subcores=NS)


def _counts_body(dst_hbm, ones_hbm, cnt_hbm, dst_v, ones_v, zbuf, acc0, acc1,
                 acc2, acc3):
    c = lax.axis_index("c")
    s = lax.axis_index("s")
    accs = [acc0, acc1, acc2, acc3]

    for i in range(ZROWS):
        zbuf[i, :] = jnp.zeros((16,), jnp.float32)
    pltpu.sync_copy(ones_hbm, ones_v)

    for el in range(4):
        @pl.loop(0, ROWS_PER_TILE // ZROWS)
        def _(i):
            pltpu.sync_copy(zbuf,
                            accs[el].at[pl.ds((s * (ROWS_PER_TILE // ZROWS) + i) * ZROWS, ZROWS)])
    plsc.subcore_barrier()

    for el in range(4):
        e = c * 4 + el
        pltpu.sync_copy(dst_hbm.at[e, s], dst_v)

        @pl.loop(0, KCH)
        def _(k):
            pltpu.sync_copy(ones_v, accs[el].at[dst_v.at[k]], add=True)
    plsc.subcore_barrier()

    for el in range(4):
        e = c * 4 + el
        pltpu.sync_copy(
            accs[el].at[pl.ds(s * ROWS_PER_TILE, ROWS_PER_TILE)],
            cnt_hbm.at[e, pl.ds(s * ROWS_PER_TILE, ROWS_PER_TILE)])


@functools.cache
def _counts_call():
    return pl.kernel(
        _counts_body,
        out_type=jax.ShapeDtypeStruct((NTYPES, R, 16), jnp.float32),
        mesh=_mesh(),
        compiler_params=pltpu.CompilerParams(use_tc_tiling_on_sc=False),
        scratch_types=[
            pltpu.VMEM((KCH, CHUNK), jnp.int32),
            pltpu.VMEM((CHUNK, 16), jnp.float32),
            pltpu.VMEM((ZROWS, 16), jnp.float32),
            pltpu.VMEM_SHARED((R, 16), jnp.float32),
            pltpu.VMEM_SHARED((R, 16), jnp.float32),
            pltpu.VMEM_SHARED((R, 16), jnp.float32),
            pltpu.VMEM_SHARED((R, 16), jnp.float32),
        ],
    )


def _segsum_body(par, x2_hbm, src_hbm, dst_hbm, seg_hbm, src_v, dst_v,
                 rows0, rows1, zbuf, sem0, sem1, zsem, acc):
    h = lax.axis_index("c")
    s = lax.axis_index("s")

    for i in range(ZROWS):
        for j in range(4):
            zbuf[i, pl.ds(j * 16, 16)] = jnp.zeros((16,), jnp.float32)

    @pl.loop(0, NTYPES // 2)
    def _(t):
        e = 2 * t + par
        base2 = jnp.where(t < 2, 0, 2 * N)

        @pl.loop(0, ROWS_PER_TILE // ZROWS)
        def _(i):
            pltpu.async_copy(
                zbuf,
                acc.at[pl.ds((s * (ROWS_PER_TILE // ZROWS) + i) * ZROWS,
                             ZROWS)], zsem)

        @pl.loop(0, ROWS_PER_TILE // ZROWS)
        def _(i):
            pltpu.make_async_copy(
                zbuf,
                acc.at[pl.ds((s * (ROWS_PER_TILE // ZROWS) + i) * ZROWS,
                             ZROWS)], zsem).wait()
        plsc.subcore_barrier()

        pltpu.sync_copy(src_hbm.at[e, s], src_v)
        pltpu.sync_copy(dst_hbm.at[e, s], dst_v)

        @pl.loop(0, KCH)
        def _(k):
            for j in range(CHUNK // 16):
                sv = src_v[k, pl.ds(j * 16, 16)]
                src_v[k, pl.ds(j * 16, 16)] = sv * 2 + (base2 + h)

        pltpu.async_copy(x2_hbm.at[src_v.at[0]], rows0, sem0)

        @pl.loop(0, KCH // 2)
        def _(p):
            k0 = 2 * p
            pltpu.async_copy(x2_hbm.at[src_v.at[k0 + 1]], rows1, sem1)
            pltpu.make_async_copy(x2_hbm.at[src_v.at[k0]], rows0, sem0).wait()
            pltpu.sync_copy(rows0, acc.at[dst_v.at[k0]], add=True)

            @pl.when(p < KCH // 2 - 1)
            def _():
                pltpu.async_copy(x2_hbm.at[src_v.at[k0 + 2]], rows0, sem0)
            pltpu.make_async_copy(
                x2_hbm.at[src_v.at[k0 + 1]], rows1, sem1).wait()
            pltpu.sync_copy(rows1, acc.at[dst_v.at[k0 + 1]], add=True)
        plsc.subcore_barrier()

        pltpu.sync_copy(
            acc.at[pl.ds(s * ROWS_PER_TILE, ROWS_PER_TILE)],
            seg_hbm.at[t, pl.ds(s * ROWS_PER_TILE, ROWS_PER_TILE),
                       pl.ds(64 * h, 64)])
        plsc.subcore_barrier()


@functools.cache
def _segsum_call(par):
    return pl.kernel(
        functools.partial(_segsum_body, par),
        out_type=jax.ShapeDtypeStruct((NTYPES // 2, R, H), jnp.float32),
        mesh=_mesh(),
        compiler_params=pltpu.CompilerParams(use_tc_tiling_on_sc=False),
        scratch_types=[
            pltpu.VMEM((KCH, CHUNK), jnp.int32),
            pltpu.VMEM((KCH, CHUNK), jnp.int32),
            pltpu.VMEM((CHUNK, 64), jnp.float32),
            pltpu.VMEM((CHUNK, 64), jnp.float32),
            pltpu.VMEM((ZROWS, 64), jnp.float32),
            pltpu.SemaphoreType.DMA,
            pltpu.SemaphoreType.DMA,
            pltpu.SemaphoreType.DMA,
            pltpu.VMEM_SHARED((R, 64), jnp.float32),
        ],
    )


def _encode_kernel(x_ref, w_ref, b_ref, o_ref):
    o_ref[...] = (
        jnp.dot(x_ref[...], w_ref[0], preferred_element_type=jnp.float32)
        + b_ref[0])


def _encode(xin, wenc, benc):
    grid = (2 * N) // 5000
    return pl.pallas_call(
        _encode_kernel,
        grid=(grid,),
        in_specs=[
            pl.BlockSpec((5000, 8), lambda i: (i, 0)),
            pl.BlockSpec((1, 8, H), lambda i: (i // 5, 0, 0)),
            pl.BlockSpec((1, 1, H), lambda i: (i // 5, 0, 0)),
        ],
        out_specs=pl.BlockSpec((5000, H), lambda i: (i, 0)),
        out_shape=jax.ShapeDtypeStruct((2 * N, H), jnp.float32),
    )(xin, wenc, benc)


def _layer_kernel(s0, s1, s2, s3, c0, c1, c2, c3, x_ref, a_ref, b_ref, bias,
                  o_ref):
    acc = jnp.dot(x_ref[...], b_ref[...], preferred_element_type=jnp.float32)
    acc = acc + bias[...]
    for j, (sr, cr) in enumerate(((s0, c0), (s1, c1), (s2, c2), (s3, c3))):
        rcnt = 1.0 / jnp.maximum(cr[0, :, 0:1], 1.0)
        acc = acc + jnp.dot(sr[0] * rcnt, a_ref[j],
                            preferred_element_type=jnp.float32)
    o_ref[...] = jnp.maximum(acc, 0.0)


def _layer_dst(seg, cnt, x, a_dt, b_dt, bias_dt, dt):
    grid = N // 1000
    seg_spec = [
        pl.BlockSpec((1, 1000, H), functools.partial(
            lambda j, i: (j, i, 0), j))
        for j in range(4)
    ]
    cnt_spec = [
        pl.BlockSpec((1, 1000, 16), functools.partial(
            lambda j, i: (2 * j + dt, i, 0), j))
        for j in range(4)
    ]
    return pl.pallas_call(
        _layer_kernel,
        grid=(grid,),
        in_specs=seg_spec + cnt_spec + [
            pl.BlockSpec((1000, H), lambda i: (dt * 25 + i, 0)),
            pl.BlockSpec((4, H, H), lambda i: (0, 0, 0)),
            pl.BlockSpec((H, H), lambda i: (0, 0)),
            pl.BlockSpec((1, H), lambda i: (0, 0)),
        ],
        out_specs=pl.BlockSpec((1000, H), lambda i: (i, 0)),
        out_shape=jax.ShapeDtypeStruct((N, H), jnp.float32),
    )(seg, seg, seg, seg, cnt, cnt, cnt, cnt, x, a_dt, b_dt, bias_dt)


def _head_kernel(x_ref, g_ref, b_ref, w1_ref, b1_ref, w2_ref, b2_ref, o_ref,
                 acc):
    i = pl.program_id(0)

    @pl.when(i == 0)
    def _():
        acc[...] = jnp.zeros_like(acc)

    acc[...] += jnp.sum(x_ref[...], axis=0, keepdims=True)

    @pl.when(i == pl.num_programs(0) - 1)
    def _():
        scene = acc[...] / jnp.float32(2 * N)
        mu = jnp.mean(scene)
        var = jnp.mean((scene - mu) ** 2)
        hv = (scene - mu) * lax.rsqrt(var + 1e-5) * g_ref[...] + b_ref[...]
        h1 = jnp.maximum(
            jnp.dot(hv, w1_ref[...], preferred_element_type=jnp.float32)
            + b1_ref[...], 0.0)
        o_ref[...] = (
            jnp.dot(h1, w2_ref[...], preferred_element_type=jnp.float32)
            + b2_ref[...])


def _head(x, ln_g, ln_b, w1t, b1, w2t, b2):
    grid = (2 * N) // 1000
    full = lambda *s: pl.BlockSpec(s, lambda i: tuple(0 for _ in s))
    return pl.pallas_call(
        _head_kernel,
        grid=(grid,),
        in_specs=[
            pl.BlockSpec((1000, H), lambda i: (i, 0)),
            full(1, H), full(1, H), full(H, H), full(1, H), full(H, H),
            full(1, H),
        ],
        out_specs=full(1, H),
        out_shape=jax.ShapeDtypeStruct((1, H), jnp.float32),
        scratch_shapes=[pltpu.VMEM((1, H), jnp.float32)],
    )(x, ln_g.reshape(1, H), ln_b.reshape(1, H), w1t, b1.reshape(1, H), w2t,
      b2.reshape(1, H))


def kernel(x_ball, x_wall, edge_index, in_w_ball, in_b_ball, in_w_wall,
           in_b_wall, lin_l_W, lin_l_b, lin_r_W, ln_g, ln_b, out1_W, out1_b,
           out2_W, out2_b):
    f32 = jnp.float32

    wall_p = jnp.pad(x_wall, ((0, 0), (0, 8 - x_wall.shape[1])))
    ball_p = jnp.pad(x_ball, ((0, 0), (0, 8 - x_ball.shape[1])))
    xin = jnp.concatenate([wall_p, ball_p], axis=0)
    wenc = jnp.stack([
        jnp.pad(in_w_wall, ((0, 0), (0, 8 - in_w_wall.shape[1]))).T,
        jnp.pad(in_w_ball, ((0, 0), (0, 8 - in_w_ball.shape[1]))).T,
    ])
    benc = jnp.stack([in_b_wall, in_b_ball]).reshape(2, 1, H)

    pad = EPAD - E
    src_p = jnp.concatenate(
        [edge_index[:, 0, :], jnp.zeros((NTYPES, pad), jnp.int32)], axis=1)
    dst_p = jnp.concatenate(
        [edge_index[:, 1, :],
         jnp.full((NTYPES, pad), DUMMY, jnp.int32)], axis=1)
    src_r = src_p.reshape(NTYPES, NS, KCH, CHUNK)
    dst_r = dst_p.reshape(NTYPES, NS, KCH, CHUNK)

    ones_rows = jnp.ones((CHUNK, 16), f32)

    a_arrs, b_arrs, bias_arrs = [], [], []
    for l in range(2):
        at = jnp.transpose(lin_l_W[l], (0, 2, 1))
        a_arrs.append(jnp.stack([at[0::2], at[1::2]]))
        rw = lin_r_W[l]
        b_arrs.append(jnp.stack([
            jnp.sum(rw[0::2], axis=0).T,
            jnp.sum(rw[1::2], axis=0).T,
        ]))
        lb = lin_l_b[l]
        bias_arrs.append(jnp.stack([
            jnp.sum(lb[0::2], axis=0),
            jnp.sum(lb[1::2], axis=0),
        ]).reshape(2, 1, H))

    cnt = _counts_call()(dst_r, ones_rows)
    x = _encode(xin, wenc, benc)
    for l in range(2):
        x2 = x.reshape(2 * N * 2, 64)
        seg_w = _segsum_call(0)(x2, src_r, dst_r)
        seg_b = _segsum_call(1)(x2, src_r, dst_r)
        new_w = _layer_dst(seg_w, cnt, x, a_arrs[l][0], b_arrs[l][0],
                           bias_arrs[l][0], 0)
        new_b = _layer_dst(seg_b, cnt, x, a_arrs[l][1], b_arrs[l][1],
                           bias_arrs[l][1], 1)
        x = jnp.concatenate([new_w, new_b], axis=0)
    out = _head(x, ln_g, ln_b, out1_W.T, out1_b, out2_W.T, out2_b)
    return out.reshape(H)


# speedup vs baseline: 1.0258x; 1.0258x over previous
import functools
import jax
import jax.numpy as jnp
from jax import lax
from jax.experimental import pallas as pl
from jax.experimental.pallas import tpu as pltpu
from jax.experimental.pallas import tpu_sc as plsc

N = 25000
E = 100000
H = 128
NTYPES = 8
NC = 2
NS = 16
CHUNK = 112
KCH = 56
EPAD = NS * KCH * CHUNK
R = 25024
ROWS_PER_TILE = R // NS
ZROWS = 34
DUMMY = N

@functools.cache
def _mesh():
    return plsc.VectorSubcoreMesh(core_axis_name="c", subcore_axis_name="s",
                                  num_cores=NC, num_subcores=NS)


def _counts_body(dst_hbm, ones_hbm, cnt_hbm, dst_v, ones_v, zbuf, acc0, acc1,
                 acc2, acc3):
    c = lax.axis_index("c")
    s = lax.axis_index("s")
    accs = [acc0, acc1, acc2, acc3]

    for i in range(ZROWS):
        zbuf[i, :] = jnp.zeros((16,), jnp.float32)
    pltpu.sync_copy(ones_hbm, ones_v)

    for el in range(4):
        @pl.loop(0, ROWS_PER_TILE // ZROWS)
        def _(i):
            pltpu.sync_copy(zbuf,
                            accs[el].at[pl.ds((s * (ROWS_PER_TILE // ZROWS) + i) * ZROWS, ZROWS)])
    plsc.subcore_barrier()

    for el in range(4):
        e = c * 4 + el
        pltpu.sync_copy(dst_hbm.at[e, s], dst_v)

        @pl.loop(0, KCH)
        def _(k):
            pltpu.sync_copy(ones_v, accs[el].at[dst_v.at[k]], add=True)
    plsc.subcore_barrier()

    for el in range(4):
        e = c * 4 + el
        pltpu.sync_copy(
            accs[el].at[pl.ds(s * ROWS_PER_TILE, ROWS_PER_TILE)],
            cnt_hbm.at[e, pl.ds(s * ROWS_PER_TILE, ROWS_PER_TILE)])


@functools.cache
def _counts_call():
    return pl.kernel(
        _counts_body,
        out_type=jax.ShapeDtypeStruct((NTYPES, R, 16), jnp.float32),
        mesh=_mesh(),
        compiler_params=pltpu.CompilerParams(use_tc_tiling_on_sc=False),
        scratch_types=[
            pltpu.VMEM((KCH, CHUNK), jnp.int32),
            pltpu.VMEM((CHUNK, 16), jnp.float32),
            pltpu.VMEM((ZROWS, 16), jnp.float32),
            pltpu.VMEM_SHARED((R, 16), jnp.float32),
            pltpu.VMEM_SHARED((R, 16), jnp.float32),
            pltpu.VMEM_SHARED((R, 16), jnp.float32),
            pltpu.VMEM_SHARED((R, 16), jnp.float32),
        ],
    )


def _segsum_body(par, x2_hbm, src_hbm, dst_hbm, seg_hbm, src_v, dst_v,
                 rows0, rows1, zbuf, sem0, sem1, zsem, acc):
    h = lax.axis_index("c")
    s = lax.axis_index("s")

    for i in range(ZROWS):
        for j in range(4):
            zbuf[i, pl.ds(j * 16, 16)] = jnp.zeros((16,), jnp.float32)

    @pl.loop(0, NTYPES // 2)
    def _(t):
        e = 2 * t + par
        base2 = jnp.where(t < 2, 0, 2 * N)

        @pl.loop(0, ROWS_PER_TILE // ZROWS)
        def _(i):
            pltpu.async_copy(
                zbuf,
                acc.at[pl.ds((s * (ROWS_PER_TILE // ZROWS) + i) * ZROWS,
                             ZROWS)], zsem)

        @pl.loop(0, ROWS_PER_TILE // ZROWS)
        def _(i):
            pltpu.make_async_copy(
                zbuf,
                acc.at[pl.ds((s * (ROWS_PER_TILE // ZROWS) + i) * ZROWS,
                             ZROWS)], zsem).wait()
        plsc.subcore_barrier()

        pltpu.sync_copy(src_hbm.at[e, s], src_v)
        pltpu.sync_copy(dst_hbm.at[e, s], dst_v)

        @pl.loop(0, KCH)
        def _(k):
            for j in range(CHUNK // 16):
                sv = src_v[k, pl.ds(j * 16, 16)]
                src_v[k, pl.ds(j * 16, 16)] = sv * 2 + (base2 + h)

        pltpu.async_copy(x2_hbm.at[src_v.at[0]], rows0, sem0)

        @pl.loop(0, KCH // 2)
        def _(p):
            k0 = 2 * p
            pltpu.async_copy(x2_hbm.at[src_v.at[k0 + 1]], rows1, sem1)
            pltpu.make_async_copy(x2_hbm.at[src_v.at[k0]], rows0, sem0).wait()
            pltpu.sync_copy(rows0, acc.at[dst_v.at[k0]], add=True)

            @pl.when(p < KCH // 2 - 1)
            def _():
                pltpu.async_copy(x2_hbm.at[src_v.at[k0 + 2]], rows0, sem0)
            pltpu.make_async_copy(
                x2_hbm.at[src_v.at[k0 + 1]], rows1, sem1).wait()
            pltpu.sync_copy(rows1, acc.at[dst_v.at[k0 + 1]], add=True)
        plsc.subcore_barrier()

        pltpu.sync_copy(
            acc.at[pl.ds(s * ROWS_PER_TILE, ROWS_PER_TILE)],
            seg_hbm.at[t, pl.ds(s * ROWS_PER_TILE, ROWS_PER_TILE),
                       pl.ds(64 * h, 64)])
        plsc.subcore_barrier()


@functools.cache
def _segsum_call(par):
    return pl.kernel(
        functools.partial(_segsum_body, par),
        out_type=jax.ShapeDtypeStruct((NTYPES // 2, R, H), jnp.float32),
        mesh=_mesh(),
        compiler_params=pltpu.CompilerParams(use_tc_tiling_on_sc=False),
        scratch_types=[
            pltpu.VMEM((KCH, CHUNK), jnp.int32),
            pltpu.VMEM((KCH, CHUNK), jnp.int32),
            pltpu.VMEM((CHUNK, 64), jnp.float32),
            pltpu.VMEM((CHUNK, 64), jnp.float32),
            pltpu.VMEM((ZROWS, 64), jnp.float32),
            pltpu.SemaphoreType.DMA,
            pltpu.SemaphoreType.DMA,
            pltpu.SemaphoreType.DMA,
            pltpu.VMEM_SHARED((R, 64), jnp.float32),
        ],
    )


def _encode_kernel(x_ref, w_ref, b_ref, o_ref):
    o_ref[...] = (
        jnp.dot(x_ref[...], w_ref[0], preferred_element_type=jnp.float32)
        + b_ref[0])


def _encode(xin, wenc, benc):
    grid = (2 * N) // 5000
    return pl.pallas_call(
        _encode_kernel,
        grid=(grid,),
        in_specs=[
            pl.BlockSpec((5000, 8), lambda i: (i, 0)),
            pl.BlockSpec((1, 8, H), lambda i: (i // 5, 0, 0)),
            pl.BlockSpec((1, 1, H), lambda i: (i // 5, 0, 0)),
        ],
        out_specs=pl.BlockSpec((5000, H), lambda i: (i, 0)),
        out_shape=jax.ShapeDtypeStruct((2 * N, H), jnp.float32),
    )(xin, wenc, benc)


def _layer_kernel(s0, s1, s2, s3, c0, c1, c2, c3, x_ref, a_ref, b_ref, bias,
                  o_ref):
    acc = jnp.dot(x_ref[...], b_ref[...], preferred_element_type=jnp.float32)
    acc = acc + bias[...]
    for j, (sr, cr) in enumerate(((s0, c0), (s1, c1), (s2, c2), (s3, c3))):
        rcnt = 1.0 / jnp.maximum(cr[0, :, 0:1], 1.0)
        acc = acc + jnp.dot(sr[0] * rcnt, a_ref[j],
                            preferred_element_type=jnp.float32)
    o_ref[...] = jnp.maximum(acc, 0.0)


def _layer_dst(seg, cnt, x, a_dt, b_dt, bias_dt, dt):
    grid = N // 1000
    seg_spec = [
        pl.BlockSpec((1, 1000, H), functools.partial(
            lambda j, i: (j, i, 0), j))
        for j in range(4)
    ]
    cnt_spec = [
        pl.BlockSpec((1, 1000, 16), functools.partial(
            lambda j, i: (2 * j + dt, i, 0), j))
        for j in range(4)
    ]
    return pl.pallas_call(
        _layer_kernel,
        grid=(grid,),
        in_specs=seg_spec + cnt_spec + [
            pl.BlockSpec((1000, H), lambda i: (dt * 25 + i, 0)),
            pl.BlockSpec((4, H, H), lambda i: (0, 0, 0)),
            pl.BlockSpec((H, H), lambda i: (0, 0)),
            pl.BlockSpec((1, H), lambda i: (0, 0)),
        ],
        out_specs=pl.BlockSpec((1000, H), lambda i: (i, 0)),
        out_shape=jax.ShapeDtypeStruct((N, H), jnp.float32),
    )(seg, seg, seg, seg, cnt, cnt, cnt, cnt, x, a_dt, b_dt, bias_dt)


def _colsum_kernel(x_ref, o_ref, acc):
    i = pl.program_id(0)

    @pl.when(i == 0)
    def _():
        acc[...] = jnp.zeros_like(acc)

    acc[...] += jnp.sum(x_ref[...], axis=0, keepdims=True)

    @pl.when(i == pl.num_programs(0) - 1)
    def _():
        o_ref[...] = acc[...]


def _colsum(x):
    full = lambda *s: pl.BlockSpec(s, lambda i: tuple(0 for _ in s))
    return pl.pallas_call(
        _colsum_kernel,
        grid=(N // 1000,),
        in_specs=[pl.BlockSpec((1000, H), lambda i: (i, 0))],
        out_specs=full(1, H),
        out_shape=jax.ShapeDtypeStruct((1, H), jnp.float32),
        scratch_shapes=[pltpu.VMEM((1, H), jnp.float32)],
    )(x)


def _head_kernel(x_ref, ws_ref, g_ref, b_ref, w1_ref, b1_ref, w2_ref, b2_ref,
                 o_ref, acc):
    i = pl.program_id(0)

    @pl.when(i == 0)
    def _():
        acc[...] = jnp.zeros_like(acc)

    acc[...] += jnp.sum(x_ref[...], axis=0, keepdims=True)

    @pl.when(i == pl.num_programs(0) - 1)
    def _():
        scene = (acc[...] + ws_ref[...]) / jnp.float32(2 * N)
        mu = jnp.mean(scene)
        var = jnp.mean((scene - mu) ** 2)
        hv = (scene - mu) * lax.rsqrt(var + 1e-5) * g_ref[...] + b_ref[...]
        h1 = jnp.maximum(
            jnp.dot(hv, w1_ref[...], preferred_element_type=jnp.float32)
            + b1_ref[...], 0.0)
        o_ref[...] = (
            jnp.dot(h1, w2_ref[...], preferred_element_type=jnp.float32)
            + b2_ref[...])


def _head(x_b, wsum, ln_g, ln_b, w1t, b1, w2t, b2):
    full = lambda *s: pl.BlockSpec(s, lambda i: tuple(0 for _ in s))
    return pl.pallas_call(
        _head_kernel,
        grid=(N // 1000,),
        in_specs=[
            pl.BlockSpec((1000, H), lambda i: (i, 0)),
            full(1, H), full(1, H), full(1, H), full(H, H), full(1, H),
            full(H, H), full(1, H),
        ],
        out_specs=full(1, H),
        out_shape=jax.ShapeDtypeStruct((1, H), jnp.float32),
        scratch_shapes=[pltpu.VMEM((1, H), jnp.float32)],
    )(x_b, wsum, ln_g.reshape(1, H), ln_b.reshape(1, H), w1t,
      b1.reshape(1, H), w2t, b2.reshape(1, H))


def kernel(x_ball, x_wall, edge_index, in_w_ball, in_b_ball, in_w_wall,
           in_b_wall, lin_l_W, lin_l_b, lin_r_W, ln_g, ln_b, out1_W, out1_b,
           out2_W, out2_b):
    f32 = jnp.float32

    wall_p = jnp.pad(x_wall, ((0, 0), (0, 8 - x_wall.shape[1])))
    ball_p = jnp.pad(x_ball, ((0, 0), (0, 8 - x_ball.shape[1])))
    xin = jnp.concatenate([wall_p, ball_p], axis=0)
    wenc = jnp.stack([
        jnp.pad(in_w_wall, ((0, 0), (0, 8 - in_w_wall.shape[1]))).T,
        jnp.pad(in_w_ball, ((0, 0), (0, 8 - in_w_ball.shape[1]))).T,
    ])
    benc = jnp.stack([in_b_wall, in_b_ball]).reshape(2, 1, H)

    pad = EPAD - E
    src_p = jnp.concatenate(
        [edge_index[:, 0, :], jnp.zeros((NTYPES, pad), jnp.int32)], axis=1)
    dst_p = jnp.concatenate(
        [edge_index[:, 1, :],
         jnp.full((NTYPES, pad), DUMMY, jnp.int32)], axis=1)
    src_r = src_p.reshape(NTYPES, NS, KCH, CHUNK)
    dst_r = dst_p.reshape(NTYPES, NS, KCH, CHUNK)

    ones_rows = jnp.ones((CHUNK, 16), f32)

    a_arrs, b_arrs, bias_arrs = [], [], []
    for l in range(2):
        at = jnp.transpose(lin_l_W[l], (0, 2, 1))
        a_arrs.append(jnp.stack([at[0::2], at[1::2]]))
        rw = lin_r_W[l]
        b_arrs.append(jnp.stack([
            jnp.sum(rw[0::2], axis=0).T,
            jnp.sum(rw[1::2], axis=0).T,
        ]))
        lb = lin_l_b[l]
        bias_arrs.append(jnp.stack([
            jnp.sum(lb[0::2], axis=0),
            jnp.sum(lb[1::2], axis=0),
        ]).reshape(2, 1, H))

    cnt = _counts_call()(dst_r, ones_rows)
    x = _encode(xin, wenc, benc)
    for l in range(2):
        x2 = x.reshape(2 * N * 2, 64)
        seg_w = _segsum_call(0)(x2, src_r, dst_r)
        seg_b = _segsum_call(1)(x2, src_r, dst_r)
        new_w = _layer_dst(seg_w, cnt, x, a_arrs[l][0], b_arrs[l][0],
                           bias_arrs[l][0], 0)
        new_b = _layer_dst(seg_b, cnt, x, a_arrs[l][1], b_arrs[l][1],
                           bias_arrs[l][1], 1)
        if l == 0:
            x = jnp.concatenate([new_w, new_b], axis=0)
    wsum = _colsum(new_w)
    out = _head(new_b, wsum, ln_g, ln_b, out1_W.T, out1_b, out2_W.T, out2_b)
    return out.reshape(H)
